# Initial kernel scaffold; baseline (speedup 1.0000x reference)
import jax
import jax.numpy as jnp
from jax.experimental import pallas as pl


def kernel(input_tensors, query):
    raise NotImplementedError("write your pallas kernel here")


# speedup vs baseline: 1.5210x; 1.5210x over previous
import jax
import jax.numpy as jnp
from jax.experimental import pallas as pl
from jax.experimental.pallas import tpu as pltpu


def _pool_kernel(x_ref, q_ref, o_ref):
    x = x_ref[...]
    q = q_ref[...]

    s = jnp.sum(x * q[jnp.newaxis, :, :], axis=-1)

    m = jnp.max(s, axis=1, keepdims=True)
    p = jnp.exp(s - m)
    l = jnp.sum(p, axis=1, keepdims=True)

    acc = jnp.sum(p[:, :, jnp.newaxis] * x, axis=1)
    o_ref[...] = (acc * pl.reciprocal(l, approx=False)).astype(o_ref.dtype)


def kernel(input_tensors, query):
    b, t, e = input_tensors.shape
    dtype = input_tensors.dtype

    bb = 16
    if b % bb != 0:
        bb = 8 if b % 8 == 0 else b

    q_row = query.reshape(1, e).astype(dtype)

    return pl.pallas_call(
        _pool_kernel,
        out_shape=jax.ShapeDtypeStruct((b, e), dtype),
        grid=(b // bb,),
        in_specs=[
            pl.BlockSpec((bb, t, e), lambda i: (i, 0, 0)),
            pl.BlockSpec((1, e), lambda i: (0, 0)),
        ],
        out_specs=pl.BlockSpec((bb, e), lambda i: (i, 0)),
        compiler_params=pltpu.CompilerParams(
            dimension_semantics=("parallel",),
            vmem_limit_bytes=100 << 20,
        ),
    )(input_tensors, q_row)


# speedup vs baseline: 1.6582x; 1.0902x over previous
import jax
import jax.numpy as jnp
from jax.experimental import pallas as pl
from jax.experimental.pallas import tpu as pltpu


def _pool_kernel(x_ref, q_ref, o_ref):
    x = x_ref[...]
    q = q_ref[...]

    s = jnp.sum(x * q[jnp.newaxis, :, :], axis=-1)

    m = jnp.max(s, axis=1, keepdims=True)
    p = jnp.exp(s - m)
    l = jnp.sum(p, axis=1, keepdims=True)

    acc = jnp.sum(p[:, :, jnp.newaxis] * x, axis=1)
    o_ref[...] = (acc * pl.reciprocal(l, approx=False)).astype(o_ref.dtype)


def kernel(input_tensors, query):
    b, t, e = input_tensors.shape
    dtype = input_tensors.dtype

    bb = 32
    if b % bb != 0:
        bb = 8 if b % 8 == 0 else b

    q_row = query.reshape(1, e).astype(dtype)

    return pl.pallas_call(
        _pool_kernel,
        out_shape=jax.ShapeDtypeStruct((b, e), dtype),
        grid=(b // bb,),
        in_specs=[
            pl.BlockSpec((bb, t, e), lambda i: (i, 0, 0)),
            pl.BlockSpec((1, e), lambda i: (0, 0)),
        ],
        out_specs=pl.BlockSpec((bb, e), lambda i: (i, 0)),
        compiler_params=pltpu.CompilerParams(
            dimension_semantics=("parallel",),
            vmem_limit_bytes=100 << 20,
        ),
    )(input_tensors, q_row)
